# Initial kernel scaffold; baseline (speedup 1.0000x reference)
import jax
import jax.numpy as jnp
from jax.experimental import pallas as pl


def kernel(x0, x1, edge_vals, edge_idx, W_xx0, W_xx1, W_yx0, W_yx1, W_yy0, W_yy1):
    raise NotImplementedError("write your pallas kernel here")


# speedup vs baseline: 4.3030x; 4.3030x over previous
import functools

import jax
import jax.numpy as jnp
from jax.experimental import pallas as pl
from jax.experimental.pallas import tpu as pltpu


def _pick_edge_chunk(E):
    for eb in (2000, 1000, 500, 100, 50, 20, 10, 5, 4, 2, 1):
        if E % eb == 0:
            return eb
    return 1


def _pick_pass_split(N):
    for P in (5, 4, 2, 1):
        if N % P:
            continue
        Np = N // P
        for bn in (512, 400, 256, 200, 128, 64, 40, 32, 16, 8):
            if Np % bn == 0:
                return P, Np, bn
    return 1, N, N


def _cgmp_kernel(src_ref, dst_ref, ev_ref, x0_ref, x1_ref,
                 wyy0_ref, wyy1_ref, wyx0_ref, wyx1_ref, wxx0_ref, wxx1_ref,
                 out0_ref, out1_ref, y0s, y1s,
                 *, C, Np, EC, NCp, BN, EB):
    i = pl.program_id(0)
    steps = EC + NCp
    p = i // steps
    r = i % steps

    @pl.when(r == 0)
    def _init():
        y0s[...] = jnp.zeros((Np, 4 * C), jnp.float32)
        y1s[...] = jnp.zeros((Np, 12 * C), jnp.float32)

    @pl.when(r < EC)
    def _edges():
        base = p * Np

        def body(e, _):
            d = dst_ref[0, 0, e]
            dl = d - base

            @pl.when((dl >= 0) & (dl < Np))
            def _():
                s = src_ref[0, 0, e]
                row0 = x0_ref[pl.ds(s, 1), :]
                row1 = x1_ref[pl.ds(s, 1), :]
                e0 = ev_ref[0, 0, e]
                e1 = ev_ref[0, 1, e]
                e2 = ev_ref[0, 2, e]
                e3 = ev_ref[0, 3, e]
                msg0 = jnp.concatenate(
                    [e0 * row0, e1 * row0, e2 * row0, e3 * row0], axis=1)
                y0s[pl.ds(dl, 1), :] = y0s[pl.ds(dl, 1), :] + msg0
                msg1 = jnp.concatenate(
                    [e0 * row1, e1 * row1, e2 * row1, e3 * row1], axis=1)
                y1s[pl.ds(dl, 1), :] = y1s[pl.ds(dl, 1), :] + msg1
            return 0

        jax.lax.fori_loop(0, EB, body, 0)

    @pl.when(r >= EC)
    def _nodes():
        j = r - EC
        lo = j * BN
        glo = p * Np + lo
        y0b = y0s[pl.ds(lo, BN), :]
        y1b = y1s[pl.ds(lo, BN), :]
        x0b = x0_ref[pl.ds(glo, BN), :]
        x1b = x1_ref[pl.ds(glo, BN), :]

        isq3 = 1.0 / jnp.sqrt(3.0)
        isq2 = 1.0 / jnp.sqrt(2.0)

        y1m = [jnp.concatenate(
                   [y1b[:, ce * 3 * C + m * C: ce * 3 * C + (m + 1) * C]
                    for ce in range(4)], axis=1) for m in range(3)]
        x1m = [x1b[:, m * C:(m + 1) * C] for m in range(3)]

        dot = functools.partial(jnp.dot, preferred_element_type=jnp.float32)

        yy0 = jnp.concatenate(
            [y0b * y0b, (y1m[0] * y1m[0] + y1m[1] * y1m[1]
                         + y1m[2] * y1m[2]) * isq3], axis=1)
        yx0 = jnp.concatenate(
            [y0b[:, :C] * x0b,
             (y1m[0][:, :C] * x1m[0] + y1m[1][:, :C] * x1m[1]
              + y1m[2][:, :C] * x1m[2]) * isq3], axis=1)
        xx0 = jnp.concatenate(
            [x0b * x0b, (x1m[0] * x1m[0] + x1m[1] * x1m[1]
                         + x1m[2] * x1m[2]) * isq3], axis=1)
        out0_ref[...] = (dot(yy0, wyy0_ref[...]) + dot(yx0, wyx0_ref[...])
                         + dot(xx0, wxx0_ref[...]))

        wyy1_q = wyy1_ref[0:4 * C, :] + wyy1_ref[4 * C:8 * C, :]
        wxx1_q = wxx1_ref[0:C, :] + wxx1_ref[C:2 * C, :]
        for m in range(3):
            m1, m2 = (m + 1) % 3, (m + 2) % 3
            q_yy = y0b * y1m[m]
            yx1 = jnp.concatenate(
                [y0b[:, :C] * x1m[m],
                 y1m[m][:, :C] * x0b,
                 (y1m[m1][:, :C] * x1m[m2]
                  - y1m[m2][:, :C] * x1m[m1]) * isq2], axis=1)
            q_xx = x0b * x1m[m]
            out1_ref[m, :, :] = (dot(q_yy, wyy1_q)
                                 + dot(yx1, wyx1_ref[...])
                                 + dot(q_xx, wxx1_q))


def kernel(x0, x1, edge_vals, edge_idx, W_xx0, W_xx1, W_yx0, W_yx1,
           W_yy0, W_yy1):
    N, C, _ = x0.shape
    E = edge_idx.shape[1]
    NE = edge_vals.shape[1]
    assert NE == 4

    x0m = x0.reshape(N, C)
    x1m = jnp.transpose(x1, (0, 2, 1)).reshape(N, 3 * C)
    src = edge_idx[0].astype(jnp.int32)
    dst = edge_idx[1].astype(jnp.int32)

    EB = _pick_edge_chunk(E)
    EC = E // EB
    P, Np, BN = _pick_pass_split(N)
    NCp = Np // BN

    src2 = src.reshape(EC, 1, EB)
    dst2 = dst.reshape(EC, 1, EB)
    ev2 = jnp.transpose(edge_vals, (1, 0)).reshape(NE, EC, EB)
    ev2 = jnp.transpose(ev2, (1, 0, 2))

    steps = EC + NCp
    grid = (P * steps,)

    def e_idx(i):
        r = i % steps
        return (jnp.where(r < EC, r, EC - 1), 0, 0)

    def ev_idx(i):
        r = i % steps
        return (jnp.where(r < EC, r, EC - 1), 0, 0)

    def out_idx(i):
        p = i // steps
        r = i % steps
        return (p * NCp + jnp.where(r < EC, 0, r - EC), 0)

    def out1_idx(i):
        p = i // steps
        r = i % steps
        return (0, p * NCp + jnp.where(r < EC, 0, r - EC), 0)

    whole = lambda shape: pl.BlockSpec(shape, lambda i: tuple(0 for _ in shape))

    res0, res1 = pl.pallas_call(
        functools.partial(_cgmp_kernel, C=C, Np=Np, EC=EC, NCp=NCp,
                          BN=BN, EB=EB),
        grid=grid,
        in_specs=[
            pl.BlockSpec((1, 1, EB), e_idx, memory_space=pltpu.SMEM),
            pl.BlockSpec((1, 1, EB), e_idx, memory_space=pltpu.SMEM),
            pl.BlockSpec((1, NE, EB), ev_idx, memory_space=pltpu.SMEM),
            whole((N, C)),
            whole((N, 3 * C)),
            whole((8 * C, C)),
            whole((12 * C, C)),
            whole((2 * C, C)),
            whole((3 * C, C)),
            whole((2 * C, C)),
            whole((3 * C, C)),
        ],
        out_specs=[
            pl.BlockSpec((BN, C), out_idx),
            pl.BlockSpec((3, BN, C), out1_idx),
        ],
        out_shape=[
            jax.ShapeDtypeStruct((N, C), jnp.float32),
            jax.ShapeDtypeStruct((3, N, C), jnp.float32),
        ],
        scratch_shapes=[
            pltpu.VMEM((Np, 4 * C), jnp.float32),
            pltpu.VMEM((Np, 12 * C), jnp.float32),
        ],
        compiler_params=pltpu.CompilerParams(
            dimension_semantics=("arbitrary",),
        ),
    )(src2, dst2, ev2, x0m, x1m, W_yy0, W_yy1, W_yx0, W_yx1, W_xx0, W_xx1)

    out0 = res0.reshape(N, C, 1)
    out1 = jnp.transpose(res1, (1, 2, 0))
    return (out0, out1)


# speedup vs baseline: 20.9208x; 4.8620x over previous
import functools

import jax
import jax.numpy as jnp
from jax import lax
from jax.experimental import pallas as pl
from jax.experimental.pallas import tpu as pltpu
from jax.experimental.pallas import tpu_sc as plsc

L = 16
NSC = 2
NSUB = 16


def _ceil_to(a, b):
    return -(-a // b) * b


def _sc_mp_kernel(x0_hbm, x1_hbm, src_hbm, dst_hbm, ev0_hbm, ev1_hbm,
                  ev2_hbm, ev3_hbm, zeros_hbm, y_hbm,
                  shard_dst, srcg, dstg, evg, x0g, x1g, msg, acc,
                  sm_ids, sm_dl,
                  sems, ssc,
                  *, C, SH, CH, RPT, P):
    c = lax.axis_index("c")
    s = lax.axis_index("s")
    R = RPT * NSUB
    W0 = 4 * C
    NCH = SH // CH
    evs = [ev0_hbm, ev1_hbm, ev2_hbm, ev3_hbm]

    pltpu.sync_copy(dst_hbm.at[pl.ds(s * SH, SH)], shard_dst)

    iota = lax.iota(jnp.int32, L)
    sbase = s * SH

    def process(cnt):
        nb = (cnt + L - 1) // L
        cntv = jnp.full((L,), cnt, jnp.int32)

        def batch(b, _):
            idsv = jnp.zeros((L,), jnp.int32)
            dlw = jnp.zeros((L,), jnp.int32)
            for e in range(L):
                q = b * L + e
                valid = q < cnt
                idq = jnp.where(valid, sm_ids[q], 0)
                dlq = jnp.where(valid, sm_dl[q], 0)
                lane = iota == e
                idsv = jnp.where(lane, jnp.full((L,), idq, jnp.int32), idsv)
                dlw = jnp.where(lane, jnp.full((L,), dlq, jnp.int32), dlw)

            cps = [pltpu.async_copy(src_hbm.at[idsv], srcg, sems.at[0])]
            for ce in range(4):
                cps.append(pltpu.async_copy(evs[ce].at[idsv], evg.at[ce],
                                            sems.at[1 + ce]))
            for cp in cps:
                cp.wait()

            validv = (b * L + iota) < cntv
            srcv = jnp.where(validv, srcg[pl.ds(0, L)], 0)
            cp0 = pltpu.async_copy(x0_hbm.at[srcv], x0g, sems.at[5])
            cp1 = pltpu.async_copy(x1_hbm.at[srcv], x1g, sems.at[6])
            cp0.wait()
            cp1.wait()

            evv = [evg[ce, pl.ds(0, L)] for ce in range(4)]
            for e in range(L):
                q = b * L + e
                valid = q < cnt
                evb = [jnp.where(valid,
                                 jnp.full((L,), evv[ce][e], jnp.float32), 0.0)
                       for ce in range(4)]
                for j in range(16):
                    if j < 4:
                        ce, get = j, (lambda k: x0g[e, pl.ds(k * L, L)])
                    else:
                        ce, m = (j - 4) // 3, (j - 4) % 3
                        get = lambda k, m=m: x1g[e, pl.ds(m * C + k * L, L)]
                    for k in range(C // L):
                        msg[j, e, pl.ds(k * L, L)] = evb[ce] * get(k)

            cps2 = []
            for j in range(16):
                cps2.append(pltpu.async_copy(msg.at[j], acc.at[dlw * 16 + j],
                                             sems.at[j % 4], add=True))
            for cp in cps2:
                cp.wait()
            return 0

        lax.fori_loop(0, nb, batch, 0)

    def one_pass(p, _):
        base = (p * NSC + c) * R
        r0 = pl.multiple_of(s * RPT * 16, 8)

        pltpu.sync_copy(zeros_hbm.at[pl.ds(0, RPT * 16)],
                        acc.at[pl.ds(r0, RPT * 16)])
        plsc.subcore_barrier()

        def chunk(ch, _):
            cbase = ch * CH

            def scan(k, cnt):
                off = cbase + k * L
                dstv = shard_dst[pl.ds(off, L)]
                dlv = dstv - base
                for i in range(L):
                    dli = dlv[i]
                    hit = (dli >= 0) & (dli < R)
                    sm_ids[cnt] = sbase + off + i
                    sm_dl[cnt] = dli
                    cnt = cnt + jnp.where(hit, 1, 0).astype(jnp.int32)
                return cnt

            cnt = lax.fori_loop(0, CH // L, scan, jnp.int32(0))

            @pl.when(cnt > 0)
            def _():
                process(cnt)
            return 0

        lax.fori_loop(0, NCH, chunk, 0)
        plsc.subcore_barrier()

        gbase = pl.multiple_of(base * 16 + r0, 8)
        pltpu.sync_copy(acc.at[pl.ds(r0, RPT * 16)],
                        y_hbm.at[pl.ds(gbase, RPT * 16)])
        return 0

    lax.fori_loop(0, P, one_pass, 0)


def _sc_message_passing(x0m, x1m, src, dst, ev, N, C):
    E = src.shape[0]
    CH = 512 if E >= 8192 else L
    SH = _ceil_to(-(-E // NSUB), CH)
    EPAD = NSUB * SH
    RPT = 32 if N >= 2048 else 8
    R = RPT * NSUB
    P = -(-N // (NSC * R))
    NPAD = P * NSC * R

    pad = EPAD - E
    srcp = jnp.pad(src, (0, pad))
    dstp = jnp.pad(dst, (0, pad), constant_values=jnp.int32(1 << 28))
    evp = [jnp.pad(ev[:, ce], (0, pad)) for ce in range(4)]
    zeros = jnp.zeros((RPT * 16, C), jnp.float32)

    mesh = plsc.VectorSubcoreMesh(core_axis_name="c", subcore_axis_name="s",
                                  num_cores=NSC, num_subcores=NSUB)
    y = pl.kernel(
        functools.partial(_sc_mp_kernel, C=C, SH=SH, CH=CH, RPT=RPT, P=P),
        out_type=jax.ShapeDtypeStruct((NPAD * 16, C), jnp.float32),
        mesh=mesh,
        scratch_types=[
            pltpu.VMEM((SH,), jnp.int32),
            pltpu.VMEM((L,), jnp.int32),
            pltpu.VMEM((L,), jnp.int32),
            pltpu.VMEM((4, L), jnp.float32),
            pltpu.VMEM((L, C), jnp.float32),
            pltpu.VMEM((L, 3 * C), jnp.float32),
            pltpu.VMEM((16, L, C), jnp.float32),
            pltpu.VMEM_SHARED((R * 16, C), jnp.float32),
            pltpu.SMEM((CH + L,), jnp.int32),
            pltpu.SMEM((CH + L,), jnp.int32),
            pltpu.SemaphoreType.DMA((7,)),
            pltpu.SMEM((4,), jnp.int32),
        ],
    )(x0m, x1m, srcp, dstp, evp[0], evp[1], evp[2], evp[3], zeros)
    return y.reshape(NPAD, 16 * C)


def _cg_lin_kernel(y_ref, x0_ref, x1_ref,
                   wyy0_ref, wyy1_ref, wyx0_ref, wyx1_ref,
                   wxx0_ref, wxx1_ref, out0_ref, out1_ref, *, C):
    y0b = y_ref[:, 0:4 * C]
    x0b = x0_ref[...]
    x1b = x1_ref[...]

    isq3 = 1.0 / jnp.sqrt(3.0)
    isq2 = 1.0 / jnp.sqrt(2.0)

    y1m = [jnp.concatenate(
               [y_ref[:, 4 * C + ce * 3 * C + m * C:
                      4 * C + ce * 3 * C + (m + 1) * C]
                for ce in range(4)], axis=1) for m in range(3)]
    x1m = [x1b[:, m * C:(m + 1) * C] for m in range(3)]

    dot = functools.partial(jnp.dot, preferred_element_type=jnp.float32)

    yy0 = jnp.concatenate(
        [y0b * y0b, (y1m[0] * y1m[0] + y1m[1] * y1m[1]
                     + y1m[2] * y1m[2]) * isq3], axis=1)
    yx0 = jnp.concatenate(
        [y0b[:, :C] * x0b,
         (y1m[0][:, :C] * x1m[0] + y1m[1][:, :C] * x1m[1]
          + y1m[2][:, :C] * x1m[2]) * isq3], axis=1)
    xx0 = jnp.concatenate(
        [x0b * x0b, (x1m[0] * x1m[0] + x1m[1] * x1m[1]
                     + x1m[2] * x1m[2]) * isq3], axis=1)
    out0_ref[...] = (dot(yy0, wyy0_ref[...]) + dot(yx0, wyx0_ref[...])
                     + dot(xx0, wxx0_ref[...]))

    wyy1_q = wyy1_ref[0:4 * C, :] + wyy1_ref[4 * C:8 * C, :]
    wxx1_q = wxx1_ref[0:C, :] + wxx1_ref[C:2 * C, :]
    for m in range(3):
        m1, m2 = (m + 1) % 3, (m + 2) % 3
        q_yy = y0b * y1m[m]
        yx1 = jnp.concatenate(
            [y0b[:, :C] * x1m[m],
             y1m[m][:, :C] * x0b,
             (y1m[m1][:, :C] * x1m[m2]
              - y1m[m2][:, :C] * x1m[m1]) * isq2], axis=1)
        q_xx = x0b * x1m[m]
        out1_ref[m, :, :] = (dot(q_yy, wyy1_q)
                             + dot(yx1, wyx1_ref[...])
                             + dot(q_xx, wxx1_q))


def _cg_linear(y, x0m, x1m, W_yy0, W_yy1, W_yx0, W_yx1, W_xx0, W_xx1, N, C):
    for bn in (400, 256, 128, 64, 40, 32, 16, 8):
        if N % bn == 0:
            BN = bn
            break
    else:
        BN = N
    grid = (N // BN,)
    whole = lambda shape: pl.BlockSpec(shape, lambda i: tuple(0 for _ in shape))
    res0, res1 = pl.pallas_call(
        functools.partial(_cg_lin_kernel, C=C),
        grid=grid,
        in_specs=[
            pl.BlockSpec((BN, 16 * C), lambda i: (i, 0)),
            pl.BlockSpec((BN, C), lambda i: (i, 0)),
            pl.BlockSpec((BN, 3 * C), lambda i: (i, 0)),
            whole((8 * C, C)),
            whole((12 * C, C)),
            whole((2 * C, C)),
            whole((3 * C, C)),
            whole((2 * C, C)),
            whole((3 * C, C)),
        ],
        out_specs=[
            pl.BlockSpec((BN, C), lambda i: (i, 0)),
            pl.BlockSpec((3, BN, C), lambda i: (0, i, 0)),
        ],
        out_shape=[
            jax.ShapeDtypeStruct((N, C), jnp.float32),
            jax.ShapeDtypeStruct((3, N, C), jnp.float32),
        ],
    )(y, x0m, x1m, W_yy0, W_yy1, W_yx0, W_yx1, W_xx0, W_xx1)
    return res0, res1


def kernel(x0, x1, edge_vals, edge_idx, W_xx0, W_xx1, W_yx0, W_yx1,
           W_yy0, W_yy1):
    N, C, _ = x0.shape
    assert edge_vals.shape[1] == 4

    x0m = x0.reshape(N, C)
    x1m = jnp.transpose(x1, (0, 2, 1)).reshape(N, 3 * C)
    src = edge_idx[0].astype(jnp.int32)
    dst = edge_idx[1].astype(jnp.int32)

    y = _sc_message_passing(x0m, x1m, src, dst, edge_vals, N, C)
    res0, res1 = _cg_linear(y[:N] if y.shape[0] != N else y,
                            x0m, x1m, W_yy0, W_yy1, W_yx0, W_yx1,
                            W_xx0, W_xx1, N, C)

    out0 = res0.reshape(N, C, 1)
    out1 = jnp.transpose(res1, (1, 2, 0))
    return (out0, out1)


# speedup vs baseline: 25.7501x; 1.2308x over previous
import functools

import jax
import jax.numpy as jnp
from jax import lax
from jax.experimental import pallas as pl
from jax.experimental.pallas import tpu as pltpu
from jax.experimental.pallas import tpu_sc as plsc

L = 16
NSC = 2
NSUB = 16


def _ceil_to(a, b):
    return -(-a // b) * b


def _sc_mp_kernel(x0_hbm, x1_hbm, src_hbm, dst_hbm, ev0_hbm, ev1_hbm,
                  ev2_hbm, ev3_hbm, zeros_hbm, y_hbm,
                  shard_dst, idxbuf, srcbuf, jdx, evg, x0g, x1g, msg, acc,
                  sm_ids, sm_dl, ssc,
                  sems,
                  *, C, SH, CH, RPT, P):
    c = lax.axis_index("c")
    s = lax.axis_index("s")
    R = RPT * NSUB
    B = L
    NCH = SH // CH
    evs = [ev0_hbm, ev1_hbm, ev2_hbm, ev3_hbm]

    pltpu.sync_copy(dst_hbm.at[pl.ds(s * SH, SH)], shard_dst)

    iota = lax.iota(jnp.int32, L)
    sbase = s * SH

    def process(cnt):
        nb = (cnt + B - 1) // B

        def batch(b, _):
            for h in range(B // L):
                idsv = jnp.zeros((L,), jnp.int32)
                dlw = jnp.zeros((L,), jnp.int32)
                for e in range(L):
                    q = b * B + h * L + e
                    valid = q < cnt
                    idq = jnp.where(valid, sm_ids[q], 0)
                    dlq = jnp.where(valid, sm_dl[q], 0)
                    lane = iota == e
                    idsv = jnp.where(lane, jnp.full((L,), idq, jnp.int32),
                                     idsv)
                    dlw = jnp.where(lane, jnp.full((L,), dlq, jnp.int32), dlw)
                idxbuf[pl.ds(h * L, L)] = idsv
                d16 = dlw * 16
                for j in range(16):
                    jdx[j, pl.ds(h * L, L)] = d16 + j

            cps = [pltpu.async_copy(src_hbm.at[idxbuf], srcbuf, sems.at[0])]
            for ce in range(4):
                cps.append(pltpu.async_copy(evs[ce].at[idxbuf], evg.at[ce],
                                            sems.at[1 + ce]))
            for cp in cps:
                cp.wait()

            cp0 = pltpu.async_copy(x0_hbm.at[srcbuf], x0g, sems.at[5])
            cp1 = pltpu.async_copy(x1_hbm.at[srcbuf], x1g, sems.at[6])
            cp0.wait()
            cp1.wait()

            evv = [[evg[ce, pl.ds(h * L, L)] for h in range(B // L)]
                   for ce in range(4)]
            for e in range(B):
                q = b * B + e
                valid = q < cnt
                evb = [jnp.where(valid,
                                 jnp.full((L,), evv[ce][e // L][e % L],
                                          jnp.float32), 0.0)
                       for ce in range(4)]
                for j in range(16):
                    if j < 4:
                        ce, get = j, (lambda k: x0g[e, pl.ds(k * L, L)])
                    else:
                        ce, m = (j - 4) // 3, (j - 4) % 3
                        get = lambda k, m=m: x1g[e, pl.ds(m * C + k * L, L)]
                    for k in range(C // L):
                        msg[j, e, pl.ds(k * L, L)] = evb[ce] * get(k)

            cps2 = []
            for j in range(16):
                cps2.append(pltpu.async_copy(msg.at[j], acc.at[jdx.at[j]],
                                             sems.at[j % 4], add=True))
            for cp in cps2:
                cp.wait()
            return 0

        lax.fori_loop(0, nb, batch, 0)

    def one_pass(p, _):
        base = (p * NSC + c) * R
        r0 = pl.multiple_of(s * RPT * 16, 8)

        pltpu.sync_copy(zeros_hbm.at[pl.ds(0, RPT * 16)],
                        acc.at[pl.ds(r0, RPT * 16)])
        plsc.subcore_barrier()

        ssc[0] = jnp.int32(0)

        def chunk(ch, _):
            cbase = ch * CH

            def scan(k, _k):
                off = cbase + k * L
                dstv = shard_dst[pl.ds(off, L)]
                dlv = dstv - base
                hitv = jnp.where((dlv >= 0) & (dlv < R), 1, 0)
                t = hitv.astype(jnp.int32)
                for st in (1, 2, 4, 8):
                    t = t | t.at[(iota + st) & (L - 1)].get(
                        mode="promise_in_bounds")

                @pl.when(t[0] > 0)
                def _():
                    cnt = ssc[0]
                    for i in range(L):
                        dli = dlv[i]
                        hit = (dli >= 0) & (dli < R)
                        sm_ids[cnt] = sbase + off + i
                        sm_dl[cnt] = dli
                        cnt = cnt + jnp.where(hit, 1, 0).astype(jnp.int32)
                    ssc[0] = cnt
                return 0

            lax.fori_loop(0, CH // L, scan, 0)
            cnt = ssc[0]
            nfull = cnt // B

            @pl.when(nfull > 0)
            def _():
                process(nfull * B)
                for i in range(B - 1):
                    sm_ids[i] = sm_ids[nfull * B + i]
                    sm_dl[i] = sm_dl[nfull * B + i]
            ssc[0] = cnt - nfull * B
            return 0

        lax.fori_loop(0, NCH, chunk, 0)
        rem = ssc[0]

        @pl.when(rem > 0)
        def _():
            process(rem)
        ssc[0] = jnp.int32(0)
        plsc.subcore_barrier()

        gbase = pl.multiple_of(base * 16 + r0, 8)
        pltpu.sync_copy(acc.at[pl.ds(r0, RPT * 16)],
                        y_hbm.at[pl.ds(gbase, RPT * 16)])
        return 0

    lax.fori_loop(0, P, one_pass, 0)


def _sc_message_passing(x0m, x1m, src, dst, ev, N, C):
    E = src.shape[0]
    CH = 512 if E >= 8192 else L
    SH = _ceil_to(-(-E // NSUB), CH)
    EPAD = NSUB * SH
    RPT = 32 if N >= 2048 else 8
    R = RPT * NSUB
    P = -(-N // (NSC * R))
    NPAD = P * NSC * R

    pad = EPAD - E
    srcp = jnp.pad(src, (0, pad))
    dstp = jnp.pad(dst, (0, pad), constant_values=jnp.int32(1 << 28))
    evp = [jnp.pad(ev[:, ce], (0, pad)) for ce in range(4)]
    zeros = jnp.zeros((RPT * 16, C), jnp.float32)

    mesh = plsc.VectorSubcoreMesh(core_axis_name="c", subcore_axis_name="s",
                                  num_cores=NSC, num_subcores=NSUB)
    y = pl.kernel(
        functools.partial(_sc_mp_kernel, C=C, SH=SH, CH=CH, RPT=RPT, P=P),
        out_type=jax.ShapeDtypeStruct((NPAD * 16, C), jnp.float32),
        mesh=mesh,
        scratch_types=[
            pltpu.VMEM((SH,), jnp.int32),
            pltpu.VMEM((L,), jnp.int32),
            pltpu.VMEM((L,), jnp.int32),
            pltpu.VMEM((16, L), jnp.int32),
            pltpu.VMEM((4, L), jnp.float32),
            pltpu.VMEM((L, C), jnp.float32),
            pltpu.VMEM((L, 3 * C), jnp.float32),
            pltpu.VMEM((16, L, C), jnp.float32),
            pltpu.VMEM_SHARED((R * 16, C), jnp.float32),
            pltpu.SMEM((CH + 2 * L,), jnp.int32),
            pltpu.SMEM((CH + 2 * L,), jnp.int32),
            pltpu.SMEM((4,), jnp.int32),
            pltpu.SemaphoreType.DMA((7,)),
        ],
    )(x0m, x1m, srcp, dstp, evp[0], evp[1], evp[2], evp[3], zeros)
    return y.reshape(NPAD, 16, C)


def _cg_lin_kernel(y_ref, x0_ref, x1_ref,
                   wyy0_ref, wyy1_ref, wyx0_ref, wyx1_ref,
                   wxx0_ref, wxx1_ref, out0_ref, out1_ref, *, C):
    y0b = jnp.concatenate([y_ref[:, ce, :] for ce in range(4)], axis=1)
    x0b = x0_ref[...]
    x1b = x1_ref[...]

    isq3 = 1.0 / jnp.sqrt(3.0)
    isq2 = 1.0 / jnp.sqrt(2.0)

    y1m = [jnp.concatenate(
               [y_ref[:, 4 + ce * 3 + m, :] for ce in range(4)], axis=1)
           for m in range(3)]
    x1m = [x1b[:, m * C:(m + 1) * C] for m in range(3)]

    dot = functools.partial(jnp.dot, preferred_element_type=jnp.float32)

    yy0 = jnp.concatenate(
        [y0b * y0b, (y1m[0] * y1m[0] + y1m[1] * y1m[1]
                     + y1m[2] * y1m[2]) * isq3], axis=1)
    yx0 = jnp.concatenate(
        [y0b[:, :C] * x0b,
         (y1m[0][:, :C] * x1m[0] + y1m[1][:, :C] * x1m[1]
          + y1m[2][:, :C] * x1m[2]) * isq3], axis=1)
    xx0 = jnp.concatenate(
        [x0b * x0b, (x1m[0] * x1m[0] + x1m[1] * x1m[1]
                     + x1m[2] * x1m[2]) * isq3], axis=1)
    out0_ref[...] = (dot(yy0, wyy0_ref[...]) + dot(yx0, wyx0_ref[...])
                     + dot(xx0, wxx0_ref[...]))

    wyy1_q = wyy1_ref[0:4 * C, :] + wyy1_ref[4 * C:8 * C, :]
    wxx1_q = wxx1_ref[0:C, :] + wxx1_ref[C:2 * C, :]
    for m in range(3):
        m1, m2 = (m + 1) % 3, (m + 2) % 3
        q_yy = y0b * y1m[m]
        yx1 = jnp.concatenate(
            [y0b[:, :C] * x1m[m],
             y1m[m][:, :C] * x0b,
             (y1m[m1][:, :C] * x1m[m2]
              - y1m[m2][:, :C] * x1m[m1]) * isq2], axis=1)
        q_xx = x0b * x1m[m]
        out1_ref[m, :, :] = (dot(q_yy, wyy1_q)
                             + dot(yx1, wyx1_ref[...])
                             + dot(q_xx, wxx1_q))


def _cg_linear(y, x0m, x1m, W_yy0, W_yy1, W_yx0, W_yx1, W_xx0, W_xx1, N, C):
    for bn in (400, 256, 128, 64, 40, 32, 16, 8):
        if N % bn == 0:
            BN = bn
            break
    else:
        BN = N
    grid = (N // BN,)
    whole = lambda shape: pl.BlockSpec(shape, lambda i: tuple(0 for _ in shape))
    res0, res1 = pl.pallas_call(
        functools.partial(_cg_lin_kernel, C=C),
        grid=grid,
        in_specs=[
            pl.BlockSpec((BN, 16, C), lambda i: (i, 0, 0)),
            pl.BlockSpec((BN, C), lambda i: (i, 0)),
            pl.BlockSpec((BN, 3 * C), lambda i: (i, 0)),
            whole((8 * C, C)),
            whole((12 * C, C)),
            whole((2 * C, C)),
            whole((3 * C, C)),
            whole((2 * C, C)),
            whole((3 * C, C)),
        ],
        out_specs=[
            pl.BlockSpec((BN, C), lambda i: (i, 0)),
            pl.BlockSpec((3, BN, C), lambda i: (0, i, 0)),
        ],
        out_shape=[
            jax.ShapeDtypeStruct((N, C), jnp.float32),
            jax.ShapeDtypeStruct((3, N, C), jnp.float32),
        ],
    )(y, x0m, x1m, W_yy0, W_yy1, W_yx0, W_yx1, W_xx0, W_xx1)
    return res0, res1


def kernel(x0, x1, edge_vals, edge_idx, W_xx0, W_xx1, W_yx0, W_yx1,
           W_yy0, W_yy1):
    N, C, _ = x0.shape
    assert edge_vals.shape[1] == 4

    x0m = x0.reshape(N, C)
    x1m = jnp.transpose(x1, (0, 2, 1)).reshape(N, 3 * C)
    src = edge_idx[0].astype(jnp.int32)
    dst = edge_idx[1].astype(jnp.int32)

    y = _sc_message_passing(x0m, x1m, src, dst, edge_vals, N, C)
    res0, res1 = _cg_linear(y[:N] if y.shape[0] != N else y,
                            x0m, x1m, W_yy0, W_yy1, W_yx0, W_yx1,
                            W_xx0, W_xx1, N, C)

    out0 = res0.reshape(N, C, 1)
    out1 = jnp.transpose(res1, (1, 2, 0))
    return (out0, out1)


# speedup vs baseline: 25.9569x; 1.0080x over previous
import functools

import jax
import jax.numpy as jnp
from jax import lax
from jax.experimental import pallas as pl
from jax.experimental.pallas import tpu as pltpu
from jax.experimental.pallas import tpu_sc as plsc

L = 16
NSC = 2
NSUB = 16


def _ceil_to(a, b):
    return -(-a // b) * b


def _sc_mp_kernel(x0_hbm, x1_hbm, src_hbm, dst_hbm, ev0_hbm, ev1_hbm,
                  ev2_hbm, ev3_hbm, zeros_hbm, y_hbm,
                  shard_dst, idxbuf, srcbuf, jdx, evg, x0g, x1g, msg, acc,
                  sm_ids, sm_dl, ssc,
                  sems,
                  *, C, SH, CH, RPT, P):
    c = lax.axis_index("c")
    s = lax.axis_index("s")
    R = RPT * NSUB
    B = L
    NCH = SH // CH
    evs = [ev0_hbm, ev1_hbm, ev2_hbm, ev3_hbm]

    pltpu.sync_copy(dst_hbm.at[pl.ds(s * SH, SH)], shard_dst)

    iota = lax.iota(jnp.int32, L)
    sbase = s * SH

    def process(cnt):
        nb = (cnt + B - 1) // B

        def batch(b, _):
            for h in range(B // L):
                idsv = jnp.zeros((L,), jnp.int32)
                dlw = jnp.zeros((L,), jnp.int32)
                for e in range(L):
                    q = b * B + h * L + e
                    valid = q < cnt
                    idq = jnp.where(valid, sm_ids[q], 0)
                    dlq = jnp.where(valid, sm_dl[q], 0)
                    lane = iota == e
                    idsv = jnp.where(lane, jnp.full((L,), idq, jnp.int32),
                                     idsv)
                    dlw = jnp.where(lane, jnp.full((L,), dlq, jnp.int32), dlw)
                idxbuf[pl.ds(h * L, L)] = idsv
                d16 = dlw * 16
                for j in range(16):
                    jdx[j, pl.ds(h * L, L)] = d16 + j

            cp_src = pltpu.async_copy(src_hbm.at[idxbuf], srcbuf, sems.at[0])
            cps_ev = [pltpu.async_copy(evs[ce].at[idxbuf], evg.at[ce],
                                       sems.at[1 + ce]) for ce in range(4)]
            cp_src.wait()
            cp0 = pltpu.async_copy(x0_hbm.at[srcbuf], x0g, sems.at[5])
            cp1 = pltpu.async_copy(x1_hbm.at[srcbuf], x1g, sems.at[6])
            for cp in cps_ev:
                cp.wait()
            cp0.wait()
            cp1.wait()

            evv = [[evg[ce, pl.ds(h * L, L)] for h in range(B // L)]
                   for ce in range(4)]
            for e in range(B):
                q = b * B + e
                valid = q < cnt
                evb = [jnp.where(valid,
                                 jnp.full((L,), evv[ce][e // L][e % L],
                                          jnp.float32), 0.0)
                       for ce in range(4)]
                for j in range(16):
                    if j < 4:
                        ce, get = j, (lambda k: x0g[e, pl.ds(k * L, L)])
                    else:
                        ce, m = (j - 4) // 3, (j - 4) % 3
                        get = lambda k, m=m: x1g[e, pl.ds(m * C + k * L, L)]
                    for k in range(C // L):
                        msg[j, e, pl.ds(k * L, L)] = evb[ce] * get(k)

            cps2 = []
            for j in range(16):
                cps2.append(pltpu.async_copy(msg.at[j], acc.at[jdx.at[j]],
                                             sems.at[j % 4], add=True))
            for cp in cps2:
                cp.wait()
            return 0

        lax.fori_loop(0, nb, batch, 0)

    def one_pass(p, _):
        base = (p * NSC + c) * R
        r0 = pl.multiple_of(s * RPT * 16, 8)

        pltpu.sync_copy(zeros_hbm.at[pl.ds(0, RPT * 16)],
                        acc.at[pl.ds(r0, RPT * 16)])
        plsc.subcore_barrier()

        ssc[0] = jnp.int32(0)

        def chunk(ch, _):
            cbase = ch * CH

            def scan(k, _k):
                off = cbase + k * L
                dstv = shard_dst[pl.ds(off, L)]
                dlv = dstv - base
                hitv = jnp.where((dlv >= 0) & (dlv < R), 1, 0)
                t = hitv.astype(jnp.int32)
                for st in (1, 2, 4, 8):
                    t = t | t.at[(iota + st) & (L - 1)].get(
                        mode="promise_in_bounds")

                @pl.when(t[0] > 0)
                def _():
                    cnt = ssc[0]
                    for i in range(L):
                        dli = dlv[i]
                        hit = (dli >= 0) & (dli < R)
                        sm_ids[cnt] = sbase + off + i
                        sm_dl[cnt] = dli
                        cnt = cnt + jnp.where(hit, 1, 0).astype(jnp.int32)
                    ssc[0] = cnt
                return 0

            lax.fori_loop(0, CH // L, scan, 0)
            cnt = ssc[0]
            nfull = cnt // B

            @pl.when(nfull > 0)
            def _():
                process(nfull * B)
                for i in range(B - 1):
                    sm_ids[i] = sm_ids[nfull * B + i]
                    sm_dl[i] = sm_dl[nfull * B + i]
            ssc[0] = cnt - nfull * B
            return 0

        lax.fori_loop(0, NCH, chunk, 0)
        rem = ssc[0]

        @pl.when(rem > 0)
        def _():
            process(rem)
        ssc[0] = jnp.int32(0)
        plsc.subcore_barrier()

        gbase = pl.multiple_of(base * 16 + r0, 8)
        pltpu.sync_copy(acc.at[pl.ds(r0, RPT * 16)],
                        y_hbm.at[pl.ds(gbase, RPT * 16)])
        return 0

    lax.fori_loop(0, P, one_pass, 0)


def _sc_message_passing(x0m, x1m, src, dst, ev, N, C):
    E = src.shape[0]
    CH = 512 if E >= 8192 else L
    SH = _ceil_to(-(-E // NSUB), CH)
    EPAD = NSUB * SH
    RPT = 32 if N >= 2048 else 8
    R = RPT * NSUB
    P = -(-N // (NSC * R))
    NPAD = P * NSC * R

    pad = EPAD - E
    srcp = jnp.pad(src, (0, pad))
    dstp = jnp.pad(dst, (0, pad), constant_values=jnp.int32(1 << 28))
    evp = [jnp.pad(ev[:, ce], (0, pad)) for ce in range(4)]
    zeros = jnp.zeros((RPT * 16, C), jnp.float32)

    mesh = plsc.VectorSubcoreMesh(core_axis_name="c", subcore_axis_name="s",
                                  num_cores=NSC, num_subcores=NSUB)
    y = pl.kernel(
        functools.partial(_sc_mp_kernel, C=C, SH=SH, CH=CH, RPT=RPT, P=P),
        out_type=jax.ShapeDtypeStruct((NPAD * 16, C), jnp.float32),
        mesh=mesh,
        scratch_types=[
            pltpu.VMEM((SH,), jnp.int32),
            pltpu.VMEM((L,), jnp.int32),
            pltpu.VMEM((L,), jnp.int32),
            pltpu.VMEM((16, L), jnp.int32),
            pltpu.VMEM((4, L), jnp.float32),
            pltpu.VMEM((L, C), jnp.float32),
            pltpu.VMEM((L, 3 * C), jnp.float32),
            pltpu.VMEM((16, L, C), jnp.float32),
            pltpu.VMEM_SHARED((R * 16, C), jnp.float32),
            pltpu.SMEM((CH + 2 * L,), jnp.int32),
            pltpu.SMEM((CH + 2 * L,), jnp.int32),
            pltpu.SMEM((4,), jnp.int32),
            pltpu.SemaphoreType.DMA((7,)),
        ],
    )(x0m, x1m, srcp, dstp, evp[0], evp[1], evp[2], evp[3], zeros)
    return y.reshape(NPAD, 16, C)


def _cg_lin_kernel(y_ref, x0_ref, x1_ref,
                   wyy0_ref, wyy1_ref, wyx0_ref, wyx1_ref,
                   wxx0_ref, wxx1_ref, out0_ref, out1_ref, *, C):
    y0b = jnp.concatenate([y_ref[:, ce, :] for ce in range(4)], axis=1)
    x0b = x0_ref[...]
    x1b = x1_ref[...]

    isq3 = 1.0 / jnp.sqrt(3.0)
    isq2 = 1.0 / jnp.sqrt(2.0)

    y1m = [jnp.concatenate(
               [y_ref[:, 4 + ce * 3 + m, :] for ce in range(4)], axis=1)
           for m in range(3)]
    x1m = [x1b[:, m * C:(m + 1) * C] for m in range(3)]

    dot = functools.partial(jnp.dot, preferred_element_type=jnp.float32)

    yy0 = jnp.concatenate(
        [y0b * y0b, (y1m[0] * y1m[0] + y1m[1] * y1m[1]
                     + y1m[2] * y1m[2]) * isq3], axis=1)
    yx0 = jnp.concatenate(
        [y0b[:, :C] * x0b,
         (y1m[0][:, :C] * x1m[0] + y1m[1][:, :C] * x1m[1]
          + y1m[2][:, :C] * x1m[2]) * isq3], axis=1)
    xx0 = jnp.concatenate(
        [x0b * x0b, (x1m[0] * x1m[0] + x1m[1] * x1m[1]
                     + x1m[2] * x1m[2]) * isq3], axis=1)
    out0_ref[...] = (dot(yy0, wyy0_ref[...]) + dot(yx0, wyx0_ref[...])
                     + dot(xx0, wxx0_ref[...]))

    wyy1_q = wyy1_ref[0:4 * C, :] + wyy1_ref[4 * C:8 * C, :]
    wxx1_q = wxx1_ref[0:C, :] + wxx1_ref[C:2 * C, :]
    for m in range(3):
        m1, m2 = (m + 1) % 3, (m + 2) % 3
        q_yy = y0b * y1m[m]
        yx1 = jnp.concatenate(
            [y0b[:, :C] * x1m[m],
             y1m[m][:, :C] * x0b,
             (y1m[m1][:, :C] * x1m[m2]
              - y1m[m2][:, :C] * x1m[m1]) * isq2], axis=1)
        q_xx = x0b * x1m[m]
        out1_ref[m, :, :] = (dot(q_yy, wyy1_q)
                             + dot(yx1, wyx1_ref[...])
                             + dot(q_xx, wxx1_q))


def _cg_linear(y, x0m, x1m, W_yy0, W_yy1, W_yx0, W_yx1, W_xx0, W_xx1, N, C):
    for bn in (400, 256, 128, 64, 40, 32, 16, 8):
        if N % bn == 0:
            BN = bn
            break
    else:
        BN = N
    grid = (N // BN,)
    whole = lambda shape: pl.BlockSpec(shape, lambda i: tuple(0 for _ in shape))
    res0, res1 = pl.pallas_call(
        functools.partial(_cg_lin_kernel, C=C),
        grid=grid,
        in_specs=[
            pl.BlockSpec((BN, 16, C), lambda i: (i, 0, 0)),
            pl.BlockSpec((BN, C), lambda i: (i, 0)),
            pl.BlockSpec((BN, 3 * C), lambda i: (i, 0)),
            whole((8 * C, C)),
            whole((12 * C, C)),
            whole((2 * C, C)),
            whole((3 * C, C)),
            whole((2 * C, C)),
            whole((3 * C, C)),
        ],
        out_specs=[
            pl.BlockSpec((BN, C), lambda i: (i, 0)),
            pl.BlockSpec((3, BN, C), lambda i: (0, i, 0)),
        ],
        out_shape=[
            jax.ShapeDtypeStruct((N, C), jnp.float32),
            jax.ShapeDtypeStruct((3, N, C), jnp.float32),
        ],
    )(y, x0m, x1m, W_yy0, W_yy1, W_yx0, W_yx1, W_xx0, W_xx1)
    return res0, res1


def kernel(x0, x1, edge_vals, edge_idx, W_xx0, W_xx1, W_yx0, W_yx1,
           W_yy0, W_yy1):
    N, C, _ = x0.shape
    assert edge_vals.shape[1] == 4

    x0m = x0.reshape(N, C)
    x1m = jnp.transpose(x1, (0, 2, 1)).reshape(N, 3 * C)
    src = edge_idx[0].astype(jnp.int32)
    dst = edge_idx[1].astype(jnp.int32)

    y = _sc_message_passing(x0m, x1m, src, dst, edge_vals, N, C)
    res0, res1 = _cg_linear(y[:N] if y.shape[0] != N else y,
                            x0m, x1m, W_yy0, W_yy1, W_yx0, W_yx1,
                            W_xx0, W_xx1, N, C)

    out0 = res0.reshape(N, C, 1)
    out1 = jnp.transpose(res1, (1, 2, 0))
    return (out0, out1)


# speedup vs baseline: 26.1683x; 1.0081x over previous
import functools

import jax
import jax.numpy as jnp
from jax import lax
from jax.experimental import pallas as pl
from jax.experimental.pallas import tpu as pltpu
from jax.experimental.pallas import tpu_sc as plsc

L = 16
NSC = 2
NSUB = 16


def _ceil_to(a, b):
    return -(-a // b) * b


def _sc_mp_kernel(x0_hbm, x1_hbm, src_hbm, dst_hbm, ev0_hbm, ev1_hbm,
                  ev2_hbm, ev3_hbm, zeros_hbm, y_hbm,
                  shard_dst, idxbuf, srcbuf, jdx, evg, x0g, x1g, msg, acc,
                  sm_ids, sm_dl, ssc,
                  sems,
                  *, C, SH, CH, RPT, P):
    c = lax.axis_index("c")
    s = lax.axis_index("s")
    R = RPT * NSUB
    B = L
    NCH = SH // CH
    evs = [ev0_hbm, ev1_hbm, ev2_hbm, ev3_hbm]

    pltpu.sync_copy(dst_hbm.at[pl.ds(s * SH, SH)], shard_dst)

    iota = lax.iota(jnp.int32, L)
    sbase = s * SH

    def process(cnt):
        nb = (cnt + B - 1) // B

        def batch(b, _):
            for h in range(B // L):
                idsv = jnp.zeros((L,), jnp.int32)
                dlw = jnp.zeros((L,), jnp.int32)
                for e in range(L):
                    q = b * B + h * L + e
                    valid = q < cnt
                    idq = jnp.where(valid, sm_ids[q], 0)
                    dlq = jnp.where(valid, sm_dl[q], 0)
                    lane = iota == e
                    idsv = jnp.where(lane, jnp.full((L,), idq, jnp.int32),
                                     idsv)
                    dlw = jnp.where(lane, jnp.full((L,), dlq, jnp.int32), dlw)
                idxbuf[pl.ds(h * L, L)] = idsv
                d16 = dlw * 16
                for j in range(16):
                    jdx[j, pl.ds(h * L, L)] = d16 + j

            cp_src = pltpu.async_copy(src_hbm.at[idxbuf], srcbuf, sems.at[0])
            cps_ev = [pltpu.async_copy(evs[ce].at[idxbuf], evg.at[ce],
                                       sems.at[1 + ce]) for ce in range(4)]
            cp_src.wait()
            cp0 = pltpu.async_copy(x0_hbm.at[srcbuf], x0g, sems.at[5])
            cp1 = pltpu.async_copy(x1_hbm.at[srcbuf], x1g, sems.at[6])
            for cp in cps_ev:
                cp.wait()
            cp0.wait()
            cp1.wait()

            evv = [[evg[ce, pl.ds(h * L, L)] for h in range(B // L)]
                   for ce in range(4)]
            for e in range(B):
                q = b * B + e
                valid = q < cnt
                evb = [jnp.where(valid,
                                 jnp.full((L,), evv[ce][e // L][e % L],
                                          jnp.float32), 0.0)
                       for ce in range(4)]
                for j in range(16):
                    if j < 4:
                        ce, get = j, (lambda k: x0g[e, pl.ds(k * L, L)])
                    else:
                        ce, m = (j - 4) // 3, (j - 4) % 3
                        get = lambda k, m=m: x1g[e, pl.ds(m * C + k * L, L)]
                    for k in range(C // L):
                        msg[j, e, pl.ds(k * L, L)] = evb[ce] * get(k)

            cps2 = []
            for j in range(16):
                cps2.append(pltpu.async_copy(msg.at[j], acc.at[jdx.at[j]],
                                             sems.at[j % 4], add=True))
            for cp in cps2:
                cp.wait()
            return 0

        lax.fori_loop(0, nb, batch, 0)

    def one_pass(p, _):
        base = (p * NSC + c) * R
        r0 = pl.multiple_of(s * RPT * 16, 8)

        pltpu.sync_copy(zeros_hbm.at[pl.ds(0, RPT * 16)],
                        acc.at[pl.ds(r0, RPT * 16)])
        plsc.subcore_barrier()

        ssc[0] = jnp.int32(0)

        def chunk(ch, _):
            cbase = ch * CH

            def scan(k, _k):
                off = cbase + k * L
                dstv = shard_dst[pl.ds(off, L)]
                dlv = dstv - base
                hitv = jnp.where((dlv >= 0) & (dlv < R), 1, 0)
                t = hitv.astype(jnp.int32)
                for st in (1, 2, 4, 8):
                    t = t | t.at[(iota + st) & (L - 1)].get(
                        mode="promise_in_bounds")

                @pl.when(t[0] > 0)
                def _():
                    cnt = ssc[0]
                    for i in range(L):
                        dli = dlv[i]
                        hit = (dli >= 0) & (dli < R)
                        sm_ids[cnt] = sbase + off + i
                        sm_dl[cnt] = dli
                        cnt = cnt + jnp.where(hit, 1, 0).astype(jnp.int32)
                    ssc[0] = cnt
                return 0

            lax.fori_loop(0, CH // L, scan, 0)
            cnt = ssc[0]
            nfull = cnt // B

            @pl.when(nfull > 0)
            def _():
                process(nfull * B)
                for i in range(B - 1):
                    sm_ids[i] = sm_ids[nfull * B + i]
                    sm_dl[i] = sm_dl[nfull * B + i]
            ssc[0] = cnt - nfull * B
            return 0

        lax.fori_loop(0, NCH, chunk, 0)
        rem = ssc[0]

        @pl.when(rem > 0)
        def _():
            process(rem)
        ssc[0] = jnp.int32(0)
        plsc.subcore_barrier()

        gbase = pl.multiple_of(base * 16 + r0, 8)
        pltpu.sync_copy(acc.at[pl.ds(r0, RPT * 16)],
                        y_hbm.at[pl.ds(gbase, RPT * 16)])
        return 0

    lax.fori_loop(0, P, one_pass, 0)


def _sc_message_passing(x0m, x1m, src, dst, ev, N, C):
    E = src.shape[0]
    CH = 512 if E >= 8192 else L
    SH = _ceil_to(-(-E // NSUB), CH)
    EPAD = NSUB * SH
    RPT = 36 if N >= 2048 else 8
    R = RPT * NSUB
    P = -(-N // (NSC * R))
    NPAD = P * NSC * R

    pad = EPAD - E
    srcp = jnp.pad(src, (0, pad))
    dstp = jnp.pad(dst, (0, pad), constant_values=jnp.int32(1 << 28))
    evp = [jnp.pad(ev[:, ce], (0, pad)) for ce in range(4)]
    zeros = jnp.zeros((RPT * 16, C), jnp.float32)

    mesh = plsc.VectorSubcoreMesh(core_axis_name="c", subcore_axis_name="s",
                                  num_cores=NSC, num_subcores=NSUB)
    y = pl.kernel(
        functools.partial(_sc_mp_kernel, C=C, SH=SH, CH=CH, RPT=RPT, P=P),
        out_type=jax.ShapeDtypeStruct((NPAD * 16, C), jnp.float32),
        mesh=mesh,
        scratch_types=[
            pltpu.VMEM((SH,), jnp.int32),
            pltpu.VMEM((L,), jnp.int32),
            pltpu.VMEM((L,), jnp.int32),
            pltpu.VMEM((16, L), jnp.int32),
            pltpu.VMEM((4, L), jnp.float32),
            pltpu.VMEM((L, C), jnp.float32),
            pltpu.VMEM((L, 3 * C), jnp.float32),
            pltpu.VMEM((16, L, C), jnp.float32),
            pltpu.VMEM_SHARED((R * 16, C), jnp.float32),
            pltpu.SMEM((CH + 2 * L,), jnp.int32),
            pltpu.SMEM((CH + 2 * L,), jnp.int32),
            pltpu.SMEM((4,), jnp.int32),
            pltpu.SemaphoreType.DMA((7,)),
        ],
    )(x0m, x1m, srcp, dstp, evp[0], evp[1], evp[2], evp[3], zeros)
    return y.reshape(NPAD, 16, C)


def _cg_lin_kernel(y_ref, x0_ref, x1_ref,
                   wyy0_ref, wyy1_ref, wyx0_ref, wyx1_ref,
                   wxx0_ref, wxx1_ref, out0_ref, out1_ref, *, C):
    y0b = jnp.concatenate([y_ref[:, ce, :] for ce in range(4)], axis=1)
    x0b = x0_ref[...]
    x1b = x1_ref[...]

    isq3 = 1.0 / jnp.sqrt(3.0)
    isq2 = 1.0 / jnp.sqrt(2.0)

    y1m = [jnp.concatenate(
               [y_ref[:, 4 + ce * 3 + m, :] for ce in range(4)], axis=1)
           for m in range(3)]
    x1m = [x1b[:, m * C:(m + 1) * C] for m in range(3)]

    dot = functools.partial(jnp.dot, preferred_element_type=jnp.float32)

    yy0 = jnp.concatenate(
        [y0b * y0b, (y1m[0] * y1m[0] + y1m[1] * y1m[1]
                     + y1m[2] * y1m[2]) * isq3], axis=1)
    yx0 = jnp.concatenate(
        [y0b[:, :C] * x0b,
         (y1m[0][:, :C] * x1m[0] + y1m[1][:, :C] * x1m[1]
          + y1m[2][:, :C] * x1m[2]) * isq3], axis=1)
    xx0 = jnp.concatenate(
        [x0b * x0b, (x1m[0] * x1m[0] + x1m[1] * x1m[1]
                     + x1m[2] * x1m[2]) * isq3], axis=1)
    out0_ref[...] = (dot(yy0, wyy0_ref[...]) + dot(yx0, wyx0_ref[...])
                     + dot(xx0, wxx0_ref[...]))

    wyy1_q = wyy1_ref[0:4 * C, :] + wyy1_ref[4 * C:8 * C, :]
    wxx1_q = wxx1_ref[0:C, :] + wxx1_ref[C:2 * C, :]
    for m in range(3):
        m1, m2 = (m + 1) % 3, (m + 2) % 3
        q_yy = y0b * y1m[m]
        yx1 = jnp.concatenate(
            [y0b[:, :C] * x1m[m],
             y1m[m][:, :C] * x0b,
             (y1m[m1][:, :C] * x1m[m2]
              - y1m[m2][:, :C] * x1m[m1]) * isq2], axis=1)
        q_xx = x0b * x1m[m]
        out1_ref[m, :, :] = (dot(q_yy, wyy1_q)
                             + dot(yx1, wyx1_ref[...])
                             + dot(q_xx, wxx1_q))


def _cg_linear(y, x0m, x1m, W_yy0, W_yy1, W_yx0, W_yx1, W_xx0, W_xx1, N, C):
    for bn in (400, 256, 128, 64, 40, 32, 16, 8):
        if N % bn == 0:
            BN = bn
            break
    else:
        BN = N
    grid = (N // BN,)
    whole = lambda shape: pl.BlockSpec(shape, lambda i: tuple(0 for _ in shape))
    res0, res1 = pl.pallas_call(
        functools.partial(_cg_lin_kernel, C=C),
        grid=grid,
        in_specs=[
            pl.BlockSpec((BN, 16, C), lambda i: (i, 0, 0)),
            pl.BlockSpec((BN, C), lambda i: (i, 0)),
            pl.BlockSpec((BN, 3 * C), lambda i: (i, 0)),
            whole((8 * C, C)),
            whole((12 * C, C)),
            whole((2 * C, C)),
            whole((3 * C, C)),
            whole((2 * C, C)),
            whole((3 * C, C)),
        ],
        out_specs=[
            pl.BlockSpec((BN, C), lambda i: (i, 0)),
            pl.BlockSpec((3, BN, C), lambda i: (0, i, 0)),
        ],
        out_shape=[
            jax.ShapeDtypeStruct((N, C), jnp.float32),
            jax.ShapeDtypeStruct((3, N, C), jnp.float32),
        ],
    )(y, x0m, x1m, W_yy0, W_yy1, W_yx0, W_yx1, W_xx0, W_xx1)
    return res0, res1


def kernel(x0, x1, edge_vals, edge_idx, W_xx0, W_xx1, W_yx0, W_yx1,
           W_yy0, W_yy1):
    N, C, _ = x0.shape
    assert edge_vals.shape[1] == 4

    x0m = x0.reshape(N, C)
    x1m = jnp.transpose(x1, (0, 2, 1)).reshape(N, 3 * C)
    src = edge_idx[0].astype(jnp.int32)
    dst = edge_idx[1].astype(jnp.int32)

    y = _sc_message_passing(x0m, x1m, src, dst, edge_vals, N, C)
    res0, res1 = _cg_linear(y[:N] if y.shape[0] != N else y,
                            x0m, x1m, W_yy0, W_yy1, W_yx0, W_yx1,
                            W_xx0, W_xx1, N, C)

    out0 = res0.reshape(N, C, 1)
    out1 = jnp.transpose(res1, (1, 2, 0))
    return (out0, out1)
